# baseline (speedup 1.0000x reference)
import functools

import jax
import jax.numpy as jnp
from jax import lax
from jax.experimental import pallas as pl
from jax.experimental.pallas import tpu as pltpu
from jax.experimental.pallas import tpu_sc as plsc

NC = 2
NS = 16
NW = NC * NS
CHUNK = 128
NSPAN = 2


def _sc_aggregate(x, src4, dst4, zeros, n_chunks):
    n_nodes, d = x.shape
    acc_rows = zeros.shape[0]
    zrows = acc_rows // NS
    span = n_chunks // NSPAN

    mesh = plsc.VectorSubcoreMesh(core_axis_name="c", subcore_axis_name="s")

    @functools.partial(
        pl.kernel,
        out_type=jax.ShapeDtypeStruct((NC, acc_rows, d), jnp.float32),
        mesh=mesh,
        scratch_types=[
            pltpu.VMEM((span, CHUNK), jnp.int32),
            pltpu.VMEM((span, CHUNK), jnp.int32),
            pltpu.VMEM((CHUNK, d), jnp.float32),
            pltpu.VMEM((CHUNK, d), jnp.float32),
            pltpu.VMEM_SHARED((acc_rows, d), jnp.float32),
            pltpu.SemaphoreType.DMA,
            pltpu.SemaphoreType.DMA,
        ],
    )
    def sc_kernel(x_hbm, src_hbm, dst_hbm, zeros_hbm, out_hbm,
                  src_v, dst_v, rows_a, rows_b, acc, sem_a, sem_b):
        c = lax.axis_index("c")
        s = lax.axis_index("s")
        wid = s * NC + c
        pltpu.sync_copy(zeros_hbm.at[pl.ds(s * zrows, zrows)],
                        acc.at[pl.ds(s * zrows, zrows)])
        plsc.subcore_barrier()

        for h in range(NSPAN):
            pltpu.sync_copy(src_hbm.at[wid, h], src_v)
            pltpu.sync_copy(dst_hbm.at[wid, h], dst_v)
            pltpu.async_copy(x_hbm.at[src_v.at[0]], rows_a, sem_a)

            def body(i, carry):
                j = 2 * i
                pltpu.make_async_copy(
                    x_hbm.at[src_v.at[j]], rows_a, sem_a).wait()
                pltpu.async_copy(x_hbm.at[src_v.at[j + 1]], rows_b, sem_b)
                pltpu.sync_copy(rows_a, acc.at[dst_v.at[j]], add=True)
                pltpu.make_async_copy(
                    x_hbm.at[src_v.at[j + 1]], rows_b, sem_b).wait()

                @pl.when(j + 2 < span)
                def _start_next():
                    pltpu.async_copy(x_hbm.at[src_v.at[j + 2]], rows_a, sem_a)

                pltpu.sync_copy(rows_b, acc.at[dst_v.at[j + 1]], add=True)
                return carry

            lax.fori_loop(0, span // 2, body, 0, unroll=False)

        plsc.subcore_barrier()
        pltpu.sync_copy(acc.at[pl.ds(s * zrows, zrows)],
                        out_hbm.at[c, pl.ds(s * zrows, zrows)])

    return sc_kernel(x, src4, dst4, zeros)


def _tc_combine_matmul(partials, W, n_nodes):
    d = W.shape[0]
    blk = 2000

    def body(p_ref, w_ref, o_ref):
        p = p_ref[...]
        ps = p[0] + p[1]
        o_ref[...] = lax.dot_general(
            ps, w_ref[...], (((1,), (1,)), ((), ())),
            preferred_element_type=jnp.float32,
            precision=lax.Precision.HIGHEST)

    return pl.pallas_call(
        body,
        grid=(n_nodes // blk,),
        in_specs=[
            pl.BlockSpec((NC, blk, d), lambda i: (0, i, 0)),
            pl.BlockSpec((d, d), lambda i: (0, 0)),
        ],
        out_specs=pl.BlockSpec((blk, d), lambda i: (i, 0)),
        out_shape=jax.ShapeDtypeStruct((n_nodes, d), jnp.float32),
    )(partials[:, :n_nodes], W)


def kernel(x, edge_index, W):
    n_nodes, d = x.shape
    e = edge_index.shape[1]
    src = edge_index[0].astype(jnp.int32)
    dst = edge_index[1].astype(jnp.int32)

    acc_rows = -(-(n_nodes + 1) // (NS * 8)) * (NS * 8)

    n_chunks = -(-e // (NW * CHUNK * 2 * NSPAN)) * 2 * NSPAN
    e_pad = NW * n_chunks * CHUNK
    if e_pad != e:
        npad = e_pad - e
        pad_src = n_nodes + (jnp.arange(npad, dtype=jnp.int32)
                             % (acc_rows - n_nodes))
        pad_dst = jnp.arange(npad, dtype=jnp.int32) % acc_rows
        src = jnp.concatenate([src, pad_src])
        dst = jnp.concatenate([dst, pad_dst])
    src4 = src.reshape(NW, NSPAN, n_chunks // NSPAN, CHUNK)
    dst4 = dst.reshape(NW, NSPAN, n_chunks // NSPAN, CHUNK)

    zeros = jnp.zeros((acc_rows, d), jnp.float32)
    x_pad = zeros.at[:n_nodes].set(x)

    partials = _sc_aggregate(x_pad, src4, dst4, zeros, n_chunks)
    return _tc_combine_matmul(partials, W, n_nodes)


# speedup vs baseline: 1.1499x; 1.1499x over previous
import functools

import jax
import jax.numpy as jnp
from jax import lax
from jax.experimental import pallas as pl
from jax.experimental.pallas import tpu as pltpu
from jax.experimental.pallas import tpu_sc as plsc

NC = 2
NS = 16
NW = NC * NS
CHUNK = 64
NBUF = 4
NSPAN = 4


def _sc_aggregate(x, src4, dst4, zeros, n_chunks):
    n_nodes, d = x.shape
    acc_rows = zeros.shape[0]
    zrows = acc_rows // NS
    span = n_chunks // NSPAN

    mesh = plsc.VectorSubcoreMesh(core_axis_name="c", subcore_axis_name="s")

    @functools.partial(
        pl.kernel,
        out_type=jax.ShapeDtypeStruct((NC, acc_rows, d), jnp.float32),
        mesh=mesh,
        scratch_types=[
            pltpu.VMEM((span, CHUNK), jnp.int32),
            pltpu.VMEM((span, CHUNK), jnp.int32),
        ] + [pltpu.VMEM((CHUNK, d), jnp.float32) for _ in range(NBUF)] + [
            pltpu.VMEM_SHARED((acc_rows, d), jnp.float32),
        ] + [pltpu.SemaphoreType.DMA for _ in range(NBUF)],
    )
    def sc_kernel(x_hbm, src_hbm, dst_hbm, zeros_hbm, out_hbm,
                  src_v, dst_v, *rest):
        rows = rest[:NBUF]
        acc = rest[NBUF]
        sems = rest[NBUF + 1:]
        c = lax.axis_index("c")
        s = lax.axis_index("s")
        wid = s * NC + c
        pltpu.sync_copy(zeros_hbm.at[pl.ds(s * zrows, zrows)],
                        acc.at[pl.ds(s * zrows, zrows)])
        plsc.subcore_barrier()

        for h in range(NSPAN):
            pltpu.sync_copy(src_hbm.at[wid, h], src_v)
            pltpu.sync_copy(dst_hbm.at[wid, h], dst_v)
            for b in range(NBUF):
                pltpu.async_copy(x_hbm.at[src_v.at[b]], rows[b], sems[b])

            def body(q, carry):
                j0 = q * NBUF
                for b in range(NBUF):
                    j = j0 + b
                    pltpu.make_async_copy(
                        x_hbm.at[src_v.at[j]], rows[b], sems[b]).wait()
                    pltpu.sync_copy(rows[b], acc.at[dst_v.at[j]], add=True)

                    @pl.when(j + NBUF < span)
                    def _start_next():
                        pltpu.async_copy(
                            x_hbm.at[src_v.at[j + NBUF]], rows[b], sems[b])
                return carry

            lax.fori_loop(0, span // NBUF, body, 0, unroll=False)

        plsc.subcore_barrier()
        pltpu.sync_copy(acc.at[pl.ds(s * zrows, zrows)],
                        out_hbm.at[c, pl.ds(s * zrows, zrows)])

    return sc_kernel(x, src4, dst4, zeros)


def _tc_combine_matmul(partials, W, n_nodes):
    d = W.shape[0]
    blk = 2000

    def body(p_ref, w_ref, o_ref):
        p = p_ref[...]
        ps = p[0] + p[1]
        o_ref[...] = lax.dot_general(
            ps, w_ref[...], (((1,), (1,)), ((), ())),
            preferred_element_type=jnp.float32,
            precision=lax.Precision.HIGHEST)

    return pl.pallas_call(
        body,
        grid=(n_nodes // blk,),
        in_specs=[
            pl.BlockSpec((NC, blk, d), lambda i: (0, i, 0)),
            pl.BlockSpec((d, d), lambda i: (0, 0)),
        ],
        out_specs=pl.BlockSpec((blk, d), lambda i: (i, 0)),
        out_shape=jax.ShapeDtypeStruct((n_nodes, d), jnp.float32),
    )(partials[:, :n_nodes], W)


def kernel(x, edge_index, W):
    n_nodes, d = x.shape
    e = edge_index.shape[1]
    src = edge_index[0].astype(jnp.int32)
    dst = edge_index[1].astype(jnp.int32)

    acc_rows = -(-(n_nodes + 1) // (NS * 8)) * (NS * 8)

    unit = NSPAN * max(NBUF, 8)
    n_chunks = -(-e // (NW * CHUNK * unit)) * unit
    e_pad = NW * n_chunks * CHUNK
    if e_pad != e:
        npad = e_pad - e
        pad_src = n_nodes + (jnp.arange(npad, dtype=jnp.int32)
                             % (acc_rows - n_nodes))
        pad_dst = jnp.arange(npad, dtype=jnp.int32) % acc_rows
        src = jnp.concatenate([src, pad_src])
        dst = jnp.concatenate([dst, pad_dst])
    src4 = src.reshape(NW, NSPAN, n_chunks // NSPAN, CHUNK)
    dst4 = dst.reshape(NW, NSPAN, n_chunks // NSPAN, CHUNK)

    zeros = jnp.zeros((acc_rows, d), jnp.float32)
    x_pad = zeros.at[:n_nodes].set(x)

    partials = _sc_aggregate(x_pad, src4, dst4, zeros, n_chunks)
    return _tc_combine_matmul(partials, W, n_nodes)
